# Initial kernel scaffold; baseline (speedup 1.0000x reference)
import jax
import jax.numpy as jnp
from jax.experimental import pallas as pl


def kernel(x, edge_index, batch, W1, b1, W2, b2, L1w, L1b, L2w, L2b):
    raise NotImplementedError("write your pallas kernel here")


# speedup vs baseline: 14.1323x; 14.1323x over previous
import functools

import jax
import jax.numpy as jnp
from jax import lax
from jax.experimental import pallas as pl
from jax.experimental.pallas import tpu as pltpu
from jax.experimental.pallas import tpu_sc as plsc

N = 10000
E = 320000
G = 64
D_IN = 128
H = 64
C = 2

NC = 2
NS = 16
NPAD = 10240
EPT = 10112
EPAD = NC * NS * EPT
CH = 128
NCHUNK = EPT // CH
ZROWS = NPAD // NS

_mesh = plsc.VectorSubcoreMesh(core_axis_name="c", subcore_axis_name="s")
_sc_params = pltpu.CompilerParams(use_tc_tiling_on_sc=False)


def _deg_body(dst_hbm, out_hbm, didx, ones_v, zbuf, acc_sh, sem):
    c = lax.axis_index("c")
    s = lax.axis_index("s")
    wid = c * NS + s
    one16 = jnp.full((16,), 1.0, jnp.float32)
    zero16 = jnp.zeros((16,), jnp.float32)

    def fill(i, _):
        ones_v[i, :] = one16
        return 0
    lax.fori_loop(0, CH, fill, 0)

    def zfill(i, _):
        zbuf[i, :] = zero16
        return 0
    lax.fori_loop(0, ZROWS, zfill, 0)

    pltpu.sync_copy(zbuf, acc_sh.at[pl.ds(s * ZROWS, ZROWS)])
    plsc.subcore_barrier()

    base = wid * EPT

    def step(k, _):
        pltpu.sync_copy(dst_hbm.at[pl.ds(base + k * CH, CH)], didx)
        pltpu.sync_copy(ones_v, acc_sh.at[didx], add=True)
        return 0
    lax.fori_loop(0, NCHUNK, step, 0)

    plsc.subcore_barrier()
    rows = pl.ds(s * ZROWS, ZROWS)
    pltpu.sync_copy(acc_sh.at[rows], zbuf)
    pltpu.sync_copy(zbuf, out_hbm.at[c, rows])


@functools.partial(jax.jit, static_argnums=())
def _sc_deg(dstp):
    k = pl.kernel(
        _deg_body,
        out_type=jax.ShapeDtypeStruct((NC, NPAD, 16), jnp.float32),
        mesh=_mesh,
        compiler_params=_sc_params,
        scratch_types=[
            pltpu.VMEM((CH,), jnp.int32),
            pltpu.VMEM((CH, 16), jnp.float32),
            pltpu.VMEM((ZROWS, 16), jnp.float32),
            pltpu.VMEM_SHARED((NPAD, 16), jnp.float32),
            pltpu.SemaphoreType.DMA,
        ],
    )
    return k(dstp)


def _agg_body(src_hbm, dst_hbm, xs_hbm, out_hbm,
              sidx, didx, gbuf, zbuf, acc_sh, sem):
    c = lax.axis_index("c")
    s = lax.axis_index("s")
    wid = c * NS + s
    zero16 = jnp.zeros((16,), jnp.float32)

    def zfill(i, _):
        for j in range(4):
            zbuf[i, pl.ds(j * 16, 16)] = zero16
        return 0
    lax.fori_loop(0, ZROWS, zfill, 0)

    pltpu.sync_copy(zbuf, acc_sh.at[pl.ds(s * ZROWS, ZROWS)])
    plsc.subcore_barrier()

    base = wid * EPT

    def step(k, _):
        pltpu.sync_copy(src_hbm.at[pl.ds(base + k * CH, CH)], sidx)
        pltpu.sync_copy(dst_hbm.at[pl.ds(base + k * CH, CH)], didx)
        pltpu.async_copy(xs_hbm.at[sidx], gbuf, sem).wait()
        pltpu.sync_copy(gbuf, acc_sh.at[didx], add=True)
        return 0
    lax.fori_loop(0, NCHUNK, step, 0)

    plsc.subcore_barrier()
    rows = pl.ds(s * ZROWS, ZROWS)
    pltpu.sync_copy(acc_sh.at[rows], zbuf)
    pltpu.sync_copy(zbuf, out_hbm.at[c, rows])


def _sc_agg(srcp, dstp, xs):
    k = pl.kernel(
        _agg_body,
        out_type=jax.ShapeDtypeStruct((NC, NPAD, H), jnp.float32),
        mesh=_mesh,
        compiler_params=_sc_params,
        scratch_types=[
            pltpu.VMEM((CH,), jnp.int32),
            pltpu.VMEM((CH,), jnp.int32),
            pltpu.VMEM((CH, H), jnp.float32),
            pltpu.VMEM((ZROWS, H), jnp.float32),
            pltpu.VMEM_SHARED((NPAD, H), jnp.float32),
            pltpu.SemaphoreType.DMA,
        ],
    )
    return k(srcp, dstp, xs)


_BLK = 512
_NBLK = NPAD // _BLK


def _tc1_body(x_ref, w1_ref, degp_ref, xp_ref, xs_ref, dinv_ref):
    deg = degp_ref[0] + degp_ref[1] + 1.0
    dinv = lax.rsqrt(jnp.maximum(deg, 1.0))
    xp = jnp.dot(x_ref[...], w1_ref[...], preferred_element_type=jnp.float32)
    xp_ref[...] = xp
    xs_ref[...] = dinv[:, 0:1] * xp
    dinv_ref[...] = dinv


def _tc1(xpad, W1, degp):
    return pl.pallas_call(
        _tc1_body,
        grid=(_NBLK,),
        in_specs=[
            pl.BlockSpec((_BLK, D_IN), lambda i: (i, 0)),
            pl.BlockSpec((D_IN, H), lambda i: (0, 0)),
            pl.BlockSpec((NC, _BLK, 16), lambda i: (0, i, 0)),
        ],
        out_specs=[
            pl.BlockSpec((_BLK, H), lambda i: (i, 0)),
            pl.BlockSpec((_BLK, H), lambda i: (i, 0)),
            pl.BlockSpec((_BLK, 16), lambda i: (i, 0)),
        ],
        out_shape=[
            jax.ShapeDtypeStruct((NPAD, H), jnp.float32),
            jax.ShapeDtypeStruct((NPAD, H), jnp.float32),
            jax.ShapeDtypeStruct((NPAD, 16), jnp.float32),
        ],
    )(xpad, W1, degp)


def _tc2_body(acc_ref, xp_ref, dinv_ref, b1_ref, w2_ref, xp2_ref, xs2_ref):
    dinv = dinv_ref[:, 0:1]
    agg = acc_ref[0] + acc_ref[1]
    h1 = jnp.maximum(dinv * agg + (dinv * dinv) * xp_ref[...] + b1_ref[...],
                     0.0)
    xp2 = jnp.dot(h1, w2_ref[...], preferred_element_type=jnp.float32)
    xp2_ref[...] = xp2
    xs2_ref[...] = dinv * xp2


def _tc2(acc1, xp1, dinv16, b1, W2):
    return pl.pallas_call(
        _tc2_body,
        grid=(_NBLK,),
        in_specs=[
            pl.BlockSpec((NC, _BLK, H), lambda i: (0, i, 0)),
            pl.BlockSpec((_BLK, H), lambda i: (i, 0)),
            pl.BlockSpec((_BLK, 16), lambda i: (i, 0)),
            pl.BlockSpec((1, H), lambda i: (0, 0)),
            pl.BlockSpec((H, H), lambda i: (0, 0)),
        ],
        out_specs=[
            pl.BlockSpec((_BLK, H), lambda i: (i, 0)),
            pl.BlockSpec((_BLK, H), lambda i: (i, 0)),
        ],
        out_shape=[
            jax.ShapeDtypeStruct((NPAD, H), jnp.float32),
            jax.ShapeDtypeStruct((NPAD, H), jnp.float32),
        ],
    )(acc1, xp1, dinv16, b1, W2)


def _tc3_body(acc_ref, xp_ref, dinv_ref, b2_ref, batch_ref,
              l1w_ref, l1b_ref, l2w_ref, l2b_ref, out_ref):
    dinv = dinv_ref[:, 0:1]
    agg = acc_ref[0] + acc_ref[1]
    h2 = jnp.maximum(dinv * agg + (dinv * dinv) * xp_ref[...] + b2_ref[...],
                     0.0)
    seg = lax.broadcasted_iota(jnp.int32, (1, G), 1)
    p = (batch_ref[...] == seg).astype(jnp.float32)
    pooled_sum = lax.dot_general(
        p, h2, (((0,), (0,)), ((), ())), preferred_element_type=jnp.float32)
    counts = jnp.sum(p, axis=0)
    pooled = pooled_sum / jnp.maximum(counts, 1.0)[:, None]
    t = jnp.maximum(
        jnp.dot(pooled, l1w_ref[...], preferred_element_type=jnp.float32)
        + l1b_ref[...], 0.0)
    logits = (jnp.dot(t, l2w_ref[...], preferred_element_type=jnp.float32)
              + l2b_ref[...])
    m = jnp.max(logits, axis=1, keepdims=True)
    lse = jnp.log(jnp.sum(jnp.exp(logits - m), axis=1, keepdims=True))
    out_ref[...] = logits - m - lse


def _tc3(acc2, xp2, dinv16, b2, batchp, L1w, L1b, L2w, L2b):
    return pl.pallas_call(
        _tc3_body,
        out_shape=jax.ShapeDtypeStruct((G, C), jnp.float32),
    )(acc2, xp2, dinv16, b2, batchp, L1w, L1b, L2w, L2b)


def kernel(x, edge_index, batch, W1, b1, W2, b2, L1w, L1b, L2w, L2b):
    pad_e = EPAD - E
    srcp = jnp.concatenate(
        [edge_index[0], jnp.full((pad_e,), N, jnp.int32)])
    dstp = jnp.concatenate(
        [edge_index[1], jnp.full((pad_e,), N, jnp.int32)])
    xpad = jnp.pad(x, ((0, NPAD - N), (0, 0)))
    batchp = jnp.concatenate(
        [batch, jnp.full((NPAD - N,), G, jnp.int32)]).reshape(NPAD, 1)

    degp = _sc_deg(dstp)
    xp1, xs1, dinv16 = _tc1(xpad, W1, degp)
    acc1 = _sc_agg(srcp, dstp, xs1)
    xp2, xs2 = _tc2(acc1, xp1, dinv16, b1.reshape(1, H), W2)
    acc2 = _sc_agg(srcp, dstp, xs2)
    return _tc3(acc2, xp2, dinv16, b2.reshape(1, H), batchp,
                L1w, L1b.reshape(1, 32), L2w, L2b.reshape(1, C))


# speedup vs baseline: 16.9978x; 1.2028x over previous
import functools

import jax
import jax.numpy as jnp
from jax import lax
from jax.experimental import pallas as pl
from jax.experimental.pallas import tpu as pltpu
from jax.experimental.pallas import tpu_sc as plsc

N = 10000
E = 320000
G = 64
D_IN = 128
H = 64
C = 2

NC = 2
NS = 16
NPAD = 10240
EPT = 10240
EPAD = NC * NS * EPT
CH = 128
NCHUNK = EPT // CH
NG = NCHUNK // 2
ZROWS = NPAD // NS

_mesh = plsc.VectorSubcoreMesh(core_axis_name="c", subcore_axis_name="s")
_sc_params = pltpu.CompilerParams(use_tc_tiling_on_sc=False)


def _deg_body(dst_hbm, out_hbm, didx, ones_v, zbuf, acc_sh, sem):
    c = lax.axis_index("c")
    s = lax.axis_index("s")
    wid = c * NS + s
    one16 = jnp.full((16,), 1.0, jnp.float32)
    zero16 = jnp.zeros((16,), jnp.float32)

    def fill(i, _):
        ones_v[i, :] = one16
        return 0
    lax.fori_loop(0, CH, fill, 0)

    def zfill(i, _):
        zbuf[i, :] = zero16
        return 0
    lax.fori_loop(0, ZROWS, zfill, 0)

    pltpu.sync_copy(zbuf, acc_sh.at[pl.ds(s * ZROWS, ZROWS)])
    pltpu.sync_copy(dst_hbm.at[wid], didx)
    plsc.subcore_barrier()

    def step(k, _):
        pltpu.sync_copy(ones_v, acc_sh.at[didx.at[k]], add=True)
        return 0
    lax.fori_loop(0, NCHUNK, step, 0)

    plsc.subcore_barrier()
    rows = pl.ds(s * ZROWS, ZROWS)
    pltpu.sync_copy(acc_sh.at[rows], zbuf)
    pltpu.sync_copy(zbuf, out_hbm.at[c, rows])


@functools.partial(jax.jit, static_argnums=())
def _sc_deg(dstp):
    k = pl.kernel(
        _deg_body,
        out_type=jax.ShapeDtypeStruct((NC, NPAD, 16), jnp.float32),
        mesh=_mesh,
        compiler_params=_sc_params,
        scratch_types=[
            pltpu.VMEM((NCHUNK, CH), jnp.int32),
            pltpu.VMEM((CH, 16), jnp.float32),
            pltpu.VMEM((ZROWS, 16), jnp.float32),
            pltpu.VMEM_SHARED((NPAD, 16), jnp.float32),
            pltpu.SemaphoreType.DMA,
        ],
    )
    return k(dstp)


def _agg_body(src_hbm, dst_hbm, xs_hbm, out_hbm,
              sidx, didx, gb0, gb1, zbuf, acc_sh, sem0, sem1):
    c = lax.axis_index("c")
    s = lax.axis_index("s")
    wid = c * NS + s
    zero16 = jnp.zeros((16,), jnp.float32)

    def zfill(i, _):
        for j in range(4):
            zbuf[i, pl.ds(j * 16, 16)] = zero16
        return 0
    lax.fori_loop(0, ZROWS, zfill, 0)

    pltpu.sync_copy(zbuf, acc_sh.at[pl.ds(s * ZROWS, ZROWS)])
    pltpu.sync_copy(src_hbm.at[wid], sidx)
    pltpu.sync_copy(dst_hbm.at[wid], didx)
    plsc.subcore_barrier()

    pltpu.async_copy(xs_hbm.at[sidx.at[0]], gb0, sem0)

    def body(g, _):
        k0 = 2 * g
        pltpu.make_async_copy(xs_hbm.at[sidx.at[k0]], gb0, sem0).wait()
        pltpu.async_copy(xs_hbm.at[sidx.at[k0 + 1]], gb1, sem1)
        pltpu.sync_copy(gb0, acc_sh.at[didx.at[k0]], add=True)
        pltpu.make_async_copy(xs_hbm.at[sidx.at[k0 + 1]], gb1, sem1).wait()

        @pl.when(g < NG - 1)
        def _():
            pltpu.async_copy(xs_hbm.at[sidx.at[k0 + 2]], gb0, sem0)

        pltpu.sync_copy(gb1, acc_sh.at[didx.at[k0 + 1]], add=True)
        return 0
    lax.fori_loop(0, NG, body, 0)

    plsc.subcore_barrier()
    rows = pl.ds(s * ZROWS, ZROWS)
    pltpu.sync_copy(acc_sh.at[rows], zbuf)
    pltpu.sync_copy(zbuf, out_hbm.at[c, rows])


def _sc_agg(srcp, dstp, xs):
    k = pl.kernel(
        _agg_body,
        out_type=jax.ShapeDtypeStruct((NC, NPAD, H), jnp.float32),
        mesh=_mesh,
        compiler_params=_sc_params,
        scratch_types=[
            pltpu.VMEM((NCHUNK, CH), jnp.int32),
            pltpu.VMEM((NCHUNK, CH), jnp.int32),
            pltpu.VMEM((CH, H), jnp.float32),
            pltpu.VMEM((CH, H), jnp.float32),
            pltpu.VMEM((ZROWS, H), jnp.float32),
            pltpu.VMEM_SHARED((NPAD, H), jnp.float32),
            pltpu.SemaphoreType.DMA,
            pltpu.SemaphoreType.DMA,
        ],
    )
    return k(srcp, dstp, xs)


_BLK = 512
_NBLK = NPAD // _BLK


def _tc0_body(x_ref, w1_ref, xp_ref):
    xp_ref[...] = jnp.dot(x_ref[...], w1_ref[...],
                          preferred_element_type=jnp.float32)


def _tc0(xpad, W1):
    return pl.pallas_call(
        _tc0_body,
        grid=(_NBLK,),
        in_specs=[
            pl.BlockSpec((_BLK, D_IN), lambda i: (i, 0)),
            pl.BlockSpec((D_IN, H), lambda i: (0, 0)),
        ],
        out_specs=pl.BlockSpec((_BLK, H), lambda i: (i, 0)),
        out_shape=jax.ShapeDtypeStruct((NPAD, H), jnp.float32),
    )(xpad, W1)


def _tc1_body(xp_ref, degp_ref, xs_ref, dinv_ref):
    deg = degp_ref[0] + degp_ref[1] + 1.0
    dinv = lax.rsqrt(jnp.maximum(deg, 1.0))
    xs_ref[...] = dinv[:, 0:1] * xp_ref[...]
    dinv_ref[...] = dinv


def _tc1(xp1, degp):
    return pl.pallas_call(
        _tc1_body,
        grid=(_NBLK,),
        in_specs=[
            pl.BlockSpec((_BLK, H), lambda i: (i, 0)),
            pl.BlockSpec((NC, _BLK, 16), lambda i: (0, i, 0)),
        ],
        out_specs=[
            pl.BlockSpec((_BLK, H), lambda i: (i, 0)),
            pl.BlockSpec((_BLK, 16), lambda i: (i, 0)),
        ],
        out_shape=[
            jax.ShapeDtypeStruct((NPAD, H), jnp.float32),
            jax.ShapeDtypeStruct((NPAD, 16), jnp.float32),
        ],
    )(xp1, degp)


def _tc2_body(acc_ref, xp_ref, dinv_ref, b1_ref, w2_ref, xp2_ref, xs2_ref):
    dinv = dinv_ref[:, 0:1]
    agg = acc_ref[0] + acc_ref[1]
    h1 = jnp.maximum(dinv * agg + (dinv * dinv) * xp_ref[...] + b1_ref[...],
                     0.0)
    xp2 = jnp.dot(h1, w2_ref[...], preferred_element_type=jnp.float32)
    xp2_ref[...] = xp2
    xs2_ref[...] = dinv * xp2


def _tc2(acc1, xp1, dinv16, b1, W2):
    return pl.pallas_call(
        _tc2_body,
        grid=(_NBLK,),
        in_specs=[
            pl.BlockSpec((NC, _BLK, H), lambda i: (0, i, 0)),
            pl.BlockSpec((_BLK, H), lambda i: (i, 0)),
            pl.BlockSpec((_BLK, 16), lambda i: (i, 0)),
            pl.BlockSpec((1, H), lambda i: (0, 0)),
            pl.BlockSpec((H, H), lambda i: (0, 0)),
        ],
        out_specs=[
            pl.BlockSpec((_BLK, H), lambda i: (i, 0)),
            pl.BlockSpec((_BLK, H), lambda i: (i, 0)),
        ],
        out_shape=[
            jax.ShapeDtypeStruct((NPAD, H), jnp.float32),
            jax.ShapeDtypeStruct((NPAD, H), jnp.float32),
        ],
    )(acc1, xp1, dinv16, b1, W2)


def _tc3_body(acc_ref, xp_ref, dinv_ref, b2_ref, batch_ref,
              l1w_ref, l1b_ref, l2w_ref, l2b_ref, out_ref):
    dinv = dinv_ref[:, 0:1]
    agg = acc_ref[0] + acc_ref[1]
    h2 = jnp.maximum(dinv * agg + (dinv * dinv) * xp_ref[...] + b2_ref[...],
                     0.0)
    seg = lax.broadcasted_iota(jnp.int32, (1, G), 1)
    p = (batch_ref[...] == seg).astype(jnp.float32)
    pooled_sum = lax.dot_general(
        p, h2, (((0,), (0,)), ((), ())), preferred_element_type=jnp.float32)
    counts = jnp.sum(p, axis=0)
    pooled = pooled_sum / jnp.maximum(counts, 1.0)[:, None]
    t = jnp.maximum(
        jnp.dot(pooled, l1w_ref[...], preferred_element_type=jnp.float32)
        + l1b_ref[...], 0.0)
    logits = (jnp.dot(t, l2w_ref[...], preferred_element_type=jnp.float32)
              + l2b_ref[...])
    m = jnp.max(logits, axis=1, keepdims=True)
    lse = jnp.log(jnp.sum(jnp.exp(logits - m), axis=1, keepdims=True))
    out_ref[...] = logits - m - lse


def _tc3(acc2, xp2, dinv16, b2, batchp, L1w, L1b, L2w, L2b):
    return pl.pallas_call(
        _tc3_body,
        out_shape=jax.ShapeDtypeStruct((G, C), jnp.float32),
    )(acc2, xp2, dinv16, b2, batchp, L1w, L1b, L2w, L2b)


def kernel(x, edge_index, batch, W1, b1, W2, b2, L1w, L1b, L2w, L2b):
    pad_e = EPAD - E
    srcp = jnp.concatenate(
        [edge_index[0], jnp.full((pad_e,), N, jnp.int32)]
    ).reshape(NC * NS, NCHUNK, CH)
    dstp = jnp.concatenate(
        [edge_index[1], jnp.full((pad_e,), N, jnp.int32)]
    ).reshape(NC * NS, NCHUNK, CH)
    xpad = jnp.pad(x, ((0, NPAD - N), (0, 0)))
    batchp = jnp.concatenate(
        [batch, jnp.full((NPAD - N,), G, jnp.int32)]).reshape(NPAD, 1)

    degp = _sc_deg(dstp)
    xp1 = _tc0(xpad, W1)
    xs1, dinv16 = _tc1(xp1, degp)
    acc1 = _sc_agg(srcp, dstp, xs1)
    xp2, xs2 = _tc2(acc1, xp1, dinv16, b1.reshape(1, H), W2)
    acc2 = _sc_agg(srcp, dstp, xs2)
    return _tc3(acc2, xp2, dinv16, b2.reshape(1, H), batchp,
                L1w, L1b.reshape(1, 32), L2w, L2b.reshape(1, C))


# speedup vs baseline: 18.7650x; 1.1040x over previous
import functools

import jax
import jax.numpy as jnp
from jax import lax
from jax.experimental import pallas as pl
from jax.experimental.pallas import tpu as pltpu
from jax.experimental.pallas import tpu_sc as plsc

N = 10000
E = 320000
G = 64
D_IN = 128
H = 64
C = 2

NC = 2
NS = 16
NPAD = 10240
EPT = 10240
EPAD = NC * NS * EPT
CH = 128
NCHUNK = EPT // CH
NG = NCHUNK // 2
ZROWS = NPAD // NS

_mesh = plsc.VectorSubcoreMesh(core_axis_name="c", subcore_axis_name="s")
_sc_params = pltpu.CompilerParams(use_tc_tiling_on_sc=False)


def _deg_body(dst_hbm, out_hbm, didx, ones_v, zbuf, acc_sh, sem):
    c = lax.axis_index("c")
    s = lax.axis_index("s")
    wid = c * NS + s
    one16 = jnp.full((16,), 1.0, jnp.float32)
    zero16 = jnp.zeros((16,), jnp.float32)

    def fill(i, _):
        ones_v[i, :] = one16
        return 0
    lax.fori_loop(0, CH, fill, 0)

    def zfill(i, _):
        zbuf[i, :] = zero16
        return 0
    lax.fori_loop(0, ZROWS, zfill, 0)

    pltpu.sync_copy(zbuf, acc_sh.at[pl.ds(s * ZROWS, ZROWS)])
    pltpu.sync_copy(dst_hbm.at[wid], didx)
    plsc.subcore_barrier()

    def step(k, _):
        pltpu.sync_copy(ones_v, acc_sh.at[didx.at[k]], add=True)
        return 0
    lax.fori_loop(0, NCHUNK, step, 0)

    plsc.subcore_barrier()
    rows = pl.ds(s * ZROWS, ZROWS)
    pltpu.sync_copy(acc_sh.at[rows], zbuf)
    pltpu.sync_copy(zbuf, out_hbm.at[c, rows])


@functools.partial(jax.jit, static_argnums=())
def _sc_deg(dstp):
    k = pl.kernel(
        _deg_body,
        out_type=jax.ShapeDtypeStruct((NC, NPAD, 16), jnp.float32),
        mesh=_mesh,
        compiler_params=_sc_params,
        scratch_types=[
            pltpu.VMEM((NCHUNK, CH), jnp.int32),
            pltpu.VMEM((CH, 16), jnp.float32),
            pltpu.VMEM((ZROWS, 16), jnp.float32),
            pltpu.VMEM_SHARED((NPAD, 16), jnp.float32),
            pltpu.SemaphoreType.DMA,
        ],
    )
    return k(dstp)


def _agg_body(src_hbm, dst_hbm, xs_hbm, out_hbm,
              sidx, didx, gb0, gb1, gb2, gb3, zbuf, acc_sh,
              sg0, sg1, sg2, sg3):
    gbs = (gb0, gb1, gb2, gb3)
    sgs = (sg0, sg1, sg2, sg3)
    c = lax.axis_index("c")
    s = lax.axis_index("s")
    wid = c * NS + s
    zero16 = jnp.zeros((16,), jnp.float32)

    def zfill(i, _):
        for j in range(4):
            zbuf[i, pl.ds(j * 16, 16)] = zero16
        return 0
    lax.fori_loop(0, CH, zfill, 0)

    def zcopy(i, _):
        pltpu.sync_copy(zbuf, acc_sh.at[pl.ds(s * ZROWS + i * CH, CH)])
        return 0
    lax.fori_loop(0, ZROWS // CH, zcopy, 0)
    pltpu.sync_copy(src_hbm.at[wid], sidx)
    pltpu.sync_copy(dst_hbm.at[wid], didx)
    plsc.subcore_barrier()

    for b in range(3):
        pltpu.async_copy(xs_hbm.at[sidx.at[b]], gbs[b], sgs[b])

    def body(g, _):
        k0 = 4 * g
        for b in range(4):
            k = k0 + b
            pltpu.make_async_copy(
                xs_hbm.at[sidx.at[k]], gbs[b], sgs[b]).wait()

            @pl.when(k + 3 < NCHUNK)
            def _():
                pltpu.async_copy(xs_hbm.at[sidx.at[k + 3]],
                                 gbs[(b + 3) % 4], sgs[(b + 3) % 4])

            pltpu.sync_copy(gbs[b], acc_sh.at[didx.at[k]], add=True)
        return 0
    lax.fori_loop(0, NCHUNK // 4, body, 0)

    plsc.subcore_barrier()

    def wb(i, _):
        rows = pl.ds(s * ZROWS + i * CH, CH)
        pltpu.sync_copy(acc_sh.at[rows], zbuf)
        pltpu.sync_copy(zbuf, out_hbm.at[c, rows])
        return 0
    lax.fori_loop(0, ZROWS // CH, wb, 0)


def _sc_agg(srcp, dstp, xs):
    k = pl.kernel(
        _agg_body,
        out_type=jax.ShapeDtypeStruct((NC, NPAD, H), jnp.float32),
        mesh=_mesh,
        compiler_params=_sc_params,
        scratch_types=[
            pltpu.VMEM((NCHUNK, CH), jnp.int32),
            pltpu.VMEM((NCHUNK, CH), jnp.int32),
            pltpu.VMEM((CH, H), jnp.float32),
            pltpu.VMEM((CH, H), jnp.float32),
            pltpu.VMEM((CH, H), jnp.float32),
            pltpu.VMEM((CH, H), jnp.float32),
            pltpu.VMEM((CH, H), jnp.float32),
            pltpu.VMEM_SHARED((NPAD, H), jnp.float32),
            pltpu.SemaphoreType.DMA,
            pltpu.SemaphoreType.DMA,
            pltpu.SemaphoreType.DMA,
            pltpu.SemaphoreType.DMA,
        ],
    )
    return k(srcp, dstp, xs)


_BLK = 512
_NBLK = NPAD // _BLK


def _tc0_body(x_ref, w1_ref, xp_ref):
    xp_ref[...] = jnp.dot(x_ref[...], w1_ref[...],
                          preferred_element_type=jnp.float32)


def _tc0(xpad, W1):
    return pl.pallas_call(
        _tc0_body,
        grid=(_NBLK,),
        in_specs=[
            pl.BlockSpec((_BLK, D_IN), lambda i: (i, 0)),
            pl.BlockSpec((D_IN, H), lambda i: (0, 0)),
        ],
        out_specs=pl.BlockSpec((_BLK, H), lambda i: (i, 0)),
        out_shape=jax.ShapeDtypeStruct((NPAD, H), jnp.float32),
    )(xpad, W1)


def _tc1_body(xp_ref, degp_ref, xs_ref, dinv_ref):
    deg = degp_ref[0] + degp_ref[1] + 1.0
    dinv = lax.rsqrt(jnp.maximum(deg, 1.0))
    xs_ref[...] = dinv[:, 0:1] * xp_ref[...]
    dinv_ref[...] = dinv


def _tc1(xp1, degp):
    return pl.pallas_call(
        _tc1_body,
        grid=(_NBLK,),
        in_specs=[
            pl.BlockSpec((_BLK, H), lambda i: (i, 0)),
            pl.BlockSpec((NC, _BLK, 16), lambda i: (0, i, 0)),
        ],
        out_specs=[
            pl.BlockSpec((_BLK, H), lambda i: (i, 0)),
            pl.BlockSpec((_BLK, 16), lambda i: (i, 0)),
        ],
        out_shape=[
            jax.ShapeDtypeStruct((NPAD, H), jnp.float32),
            jax.ShapeDtypeStruct((NPAD, 16), jnp.float32),
        ],
    )(xp1, degp)


def _tc2_body(acc_ref, xp_ref, dinv_ref, b1_ref, w2_ref, xp2_ref, xs2_ref):
    dinv = dinv_ref[:, 0:1]
    agg = acc_ref[0] + acc_ref[1]
    h1 = jnp.maximum(dinv * agg + (dinv * dinv) * xp_ref[...] + b1_ref[...],
                     0.0)
    xp2 = jnp.dot(h1, w2_ref[...], preferred_element_type=jnp.float32)
    xp2_ref[...] = xp2
    xs2_ref[...] = dinv * xp2


def _tc2(acc1, xp1, dinv16, b1, W2):
    return pl.pallas_call(
        _tc2_body,
        grid=(_NBLK,),
        in_specs=[
            pl.BlockSpec((NC, _BLK, H), lambda i: (0, i, 0)),
            pl.BlockSpec((_BLK, H), lambda i: (i, 0)),
            pl.BlockSpec((_BLK, 16), lambda i: (i, 0)),
            pl.BlockSpec((1, H), lambda i: (0, 0)),
            pl.BlockSpec((H, H), lambda i: (0, 0)),
        ],
        out_specs=[
            pl.BlockSpec((_BLK, H), lambda i: (i, 0)),
            pl.BlockSpec((_BLK, H), lambda i: (i, 0)),
        ],
        out_shape=[
            jax.ShapeDtypeStruct((NPAD, H), jnp.float32),
            jax.ShapeDtypeStruct((NPAD, H), jnp.float32),
        ],
    )(acc1, xp1, dinv16, b1, W2)


def _tc3_body(acc_ref, xp_ref, dinv_ref, b2_ref, batch_ref,
              l1w_ref, l1b_ref, l2w_ref, l2b_ref, out_ref):
    dinv = dinv_ref[:, 0:1]
    agg = acc_ref[0] + acc_ref[1]
    h2 = jnp.maximum(dinv * agg + (dinv * dinv) * xp_ref[...] + b2_ref[...],
                     0.0)
    seg = lax.broadcasted_iota(jnp.int32, (1, G), 1)
    p = (batch_ref[...] == seg).astype(jnp.float32)
    pooled_sum = lax.dot_general(
        p, h2, (((0,), (0,)), ((), ())), preferred_element_type=jnp.float32)
    counts = jnp.sum(p, axis=0)
    pooled = pooled_sum / jnp.maximum(counts, 1.0)[:, None]
    t = jnp.maximum(
        jnp.dot(pooled, l1w_ref[...], preferred_element_type=jnp.float32)
        + l1b_ref[...], 0.0)
    logits = (jnp.dot(t, l2w_ref[...], preferred_element_type=jnp.float32)
              + l2b_ref[...])
    m = jnp.max(logits, axis=1, keepdims=True)
    lse = jnp.log(jnp.sum(jnp.exp(logits - m), axis=1, keepdims=True))
    out_ref[...] = logits - m - lse


def _tc3(acc2, xp2, dinv16, b2, batchp, L1w, L1b, L2w, L2b):
    return pl.pallas_call(
        _tc3_body,
        out_shape=jax.ShapeDtypeStruct((G, C), jnp.float32),
    )(acc2, xp2, dinv16, b2, batchp, L1w, L1b, L2w, L2b)


def kernel(x, edge_index, batch, W1, b1, W2, b2, L1w, L1b, L2w, L2b):
    pad_e = EPAD - E
    srcp = jnp.concatenate(
        [edge_index[0], jnp.full((pad_e,), N, jnp.int32)]
    ).reshape(NC * NS, NCHUNK, CH)
    dstp = jnp.concatenate(
        [edge_index[1], jnp.full((pad_e,), N, jnp.int32)]
    ).reshape(NC * NS, NCHUNK, CH)
    xpad = jnp.pad(x, ((0, NPAD - N), (0, 0)))
    batchp = jnp.concatenate(
        [batch, jnp.full((NPAD - N,), G, jnp.int32)]).reshape(NPAD, 1)

    degp = _sc_deg(dstp)
    xp1 = _tc0(xpad, W1)
    xs1, dinv16 = _tc1(xp1, degp)
    acc1 = _sc_agg(srcp, dstp, xs1)
    xp2, xs2 = _tc2(acc1, xp1, dinv16, b1.reshape(1, H), W2)
    acc2 = _sc_agg(srcp, dstp, xs2)
    return _tc3(acc2, xp2, dinv16, b2.reshape(1, H), batchp,
                L1w, L1b.reshape(1, 32), L2w, L2b.reshape(1, C))


# speedup vs baseline: 19.0351x; 1.0144x over previous
import functools

import jax
import jax.numpy as jnp
from jax import lax
from jax.experimental import pallas as pl
from jax.experimental.pallas import tpu as pltpu
from jax.experimental.pallas import tpu_sc as plsc

N = 10000
E = 320000
G = 64
D_IN = 128
H = 64
C = 2

NC = 2
NS = 16
NPAD = 10240
EPT = 10240
EPAD = NC * NS * EPT
CH = 128
NCHUNK = EPT // CH
TOTCH = EPAD // CH
N0 = 128
N1 = TOTCH // NS - N0
ZROWS = NPAD // NS

_mesh = plsc.VectorSubcoreMesh(core_axis_name="c", subcore_axis_name="s")
_sc_params = pltpu.CompilerParams(use_tc_tiling_on_sc=False)


def _deg_body(dst_hbm, out_hbm, didx, ones_v, zbuf, acc_sh, sem):
    c = lax.axis_index("c")
    s = lax.axis_index("s")
    wid = c * NS + s
    one16 = jnp.full((16,), 1.0, jnp.float32)
    zero16 = jnp.zeros((16,), jnp.float32)

    def fill(i, _):
        ones_v[i, :] = one16
        return 0
    lax.fori_loop(0, CH, fill, 0)

    def zfill(i, _):
        zbuf[i, :] = zero16
        return 0
    lax.fori_loop(0, ZROWS, zfill, 0)

    pltpu.sync_copy(zbuf, acc_sh.at[pl.ds(s * ZROWS, ZROWS)])
    pltpu.sync_copy(dst_hbm.at[pl.ds(wid * NCHUNK, NCHUNK)], didx)
    plsc.subcore_barrier()

    def step(k, _):
        pltpu.sync_copy(ones_v, acc_sh.at[didx.at[k]], add=True)
        return 0
    lax.fori_loop(0, NCHUNK, step, 0)

    plsc.subcore_barrier()
    rows = pl.ds(s * ZROWS, ZROWS)
    pltpu.sync_copy(acc_sh.at[rows], zbuf)
    pltpu.sync_copy(zbuf, out_hbm.at[c, rows])


@functools.partial(jax.jit, static_argnums=())
def _sc_deg(dstp):
    k = pl.kernel(
        _deg_body,
        out_type=jax.ShapeDtypeStruct((NC, NPAD, 16), jnp.float32),
        mesh=_mesh,
        compiler_params=_sc_params,
        scratch_types=[
            pltpu.VMEM((NCHUNK, CH), jnp.int32),
            pltpu.VMEM((CH, 16), jnp.float32),
            pltpu.VMEM((ZROWS, 16), jnp.float32),
            pltpu.VMEM_SHARED((NPAD, 16), jnp.float32),
            pltpu.SemaphoreType.DMA,
        ],
    )
    return k(dstp)


def _agg_body(src_hbm, dst_hbm, xs_hbm, out_hbm,
              sidx, didx, gb0, gb1, gb2, gb3, zbuf, acc_sh,
              sg0, sg1, sg2, sg3):
    gbs = (gb0, gb1, gb2, gb3)
    sgs = (sg0, sg1, sg2, sg3)
    c = lax.axis_index("c")
    s = lax.axis_index("s")
    wid = c * NS + s
    zero16 = jnp.zeros((16,), jnp.float32)

    def zfill(i, _):
        for j in range(4):
            zbuf[i, pl.ds(j * 16, 16)] = zero16
        return 0
    lax.fori_loop(0, CH, zfill, 0)

    def zcopy(i, _):
        pltpu.sync_copy(zbuf, acc_sh.at[pl.ds(s * ZROWS + i * CH, CH)])
        return 0
    lax.fori_loop(0, ZROWS // CH, zcopy, 0)

    def phase(cb, n):
        pltpu.sync_copy(src_hbm.at[pl.ds(cb, n)], sidx.at[pl.ds(0, n)])
        pltpu.sync_copy(dst_hbm.at[pl.ds(cb, n)], didx.at[pl.ds(0, n)])
        plsc.subcore_barrier()

        for b in range(3):
            pltpu.async_copy(xs_hbm.at[sidx.at[b]], gbs[b], sgs[b])

        def body(g, _):
            k0 = 4 * g
            for b in range(4):
                k = k0 + b
                pltpu.make_async_copy(
                    xs_hbm.at[sidx.at[k]], gbs[b], sgs[b]).wait()

                @pl.when(k + 3 < n)
                def _():
                    pltpu.async_copy(xs_hbm.at[sidx.at[k + 3]],
                                     gbs[(b + 3) % 4], sgs[(b + 3) % 4])

                pltpu.sync_copy(gbs[b], acc_sh.at[didx.at[k]], add=True)
            return 0
        lax.fori_loop(0, n // 4, body, 0)

    @pl.when(c == 0)
    def _():
        phase(s * N0, N0)

    @pl.when(c == 1)
    def _():
        phase(NS * N0 + s * N1, N1)

    plsc.subcore_barrier()

    def wb(i, _):
        rows = pl.ds(s * ZROWS + i * CH, CH)
        pltpu.sync_copy(acc_sh.at[rows], zbuf)
        pltpu.sync_copy(zbuf, out_hbm.at[c, rows])
        return 0
    lax.fori_loop(0, ZROWS // CH, wb, 0)


def _sc_agg(srcp, dstp, xs):
    k = pl.kernel(
        _agg_body,
        out_type=jax.ShapeDtypeStruct((NC, NPAD, H), jnp.float32),
        mesh=_mesh,
        compiler_params=_sc_params,
        scratch_types=[
            pltpu.VMEM((N0, CH), jnp.int32),
            pltpu.VMEM((N0, CH), jnp.int32),
            pltpu.VMEM((CH, H), jnp.float32),
            pltpu.VMEM((CH, H), jnp.float32),
            pltpu.VMEM((CH, H), jnp.float32),
            pltpu.VMEM((CH, H), jnp.float32),
            pltpu.VMEM((CH, H), jnp.float32),
            pltpu.VMEM_SHARED((NPAD, H), jnp.float32),
            pltpu.SemaphoreType.DMA,
            pltpu.SemaphoreType.DMA,
            pltpu.SemaphoreType.DMA,
            pltpu.SemaphoreType.DMA,
        ],
    )
    return k(srcp, dstp, xs)


_BLK = 512
_NBLK = NPAD // _BLK


def _tc0_body(x_ref, w1_ref, xp_ref):
    xp_ref[...] = jnp.dot(x_ref[...], w1_ref[...],
                          preferred_element_type=jnp.float32)


def _tc0(xpad, W1):
    return pl.pallas_call(
        _tc0_body,
        grid=(_NBLK,),
        in_specs=[
            pl.BlockSpec((_BLK, D_IN), lambda i: (i, 0)),
            pl.BlockSpec((D_IN, H), lambda i: (0, 0)),
        ],
        out_specs=pl.BlockSpec((_BLK, H), lambda i: (i, 0)),
        out_shape=jax.ShapeDtypeStruct((NPAD, H), jnp.float32),
    )(xpad, W1)


def _tc1_body(xp_ref, degp_ref, xs_ref, dinv_ref):
    deg = degp_ref[0] + degp_ref[1] + 1.0
    dinv = lax.rsqrt(jnp.maximum(deg, 1.0))
    xs_ref[...] = dinv[:, 0:1] * xp_ref[...]
    dinv_ref[...] = dinv


def _tc1(xp1, degp):
    return pl.pallas_call(
        _tc1_body,
        grid=(_NBLK,),
        in_specs=[
            pl.BlockSpec((_BLK, H), lambda i: (i, 0)),
            pl.BlockSpec((NC, _BLK, 16), lambda i: (0, i, 0)),
        ],
        out_specs=[
            pl.BlockSpec((_BLK, H), lambda i: (i, 0)),
            pl.BlockSpec((_BLK, 16), lambda i: (i, 0)),
        ],
        out_shape=[
            jax.ShapeDtypeStruct((NPAD, H), jnp.float32),
            jax.ShapeDtypeStruct((NPAD, 16), jnp.float32),
        ],
    )(xp1, degp)


def _tc2_body(acc_ref, xp_ref, dinv_ref, b1_ref, w2_ref, xp2_ref, xs2_ref):
    dinv = dinv_ref[:, 0:1]
    agg = acc_ref[0] + acc_ref[1]
    h1 = jnp.maximum(dinv * agg + (dinv * dinv) * xp_ref[...] + b1_ref[...],
                     0.0)
    xp2 = jnp.dot(h1, w2_ref[...], preferred_element_type=jnp.float32)
    xp2_ref[...] = xp2
    xs2_ref[...] = dinv * xp2


def _tc2(acc1, xp1, dinv16, b1, W2):
    return pl.pallas_call(
        _tc2_body,
        grid=(_NBLK,),
        in_specs=[
            pl.BlockSpec((NC, _BLK, H), lambda i: (0, i, 0)),
            pl.BlockSpec((_BLK, H), lambda i: (i, 0)),
            pl.BlockSpec((_BLK, 16), lambda i: (i, 0)),
            pl.BlockSpec((1, H), lambda i: (0, 0)),
            pl.BlockSpec((H, H), lambda i: (0, 0)),
        ],
        out_specs=[
            pl.BlockSpec((_BLK, H), lambda i: (i, 0)),
            pl.BlockSpec((_BLK, H), lambda i: (i, 0)),
        ],
        out_shape=[
            jax.ShapeDtypeStruct((NPAD, H), jnp.float32),
            jax.ShapeDtypeStruct((NPAD, H), jnp.float32),
        ],
    )(acc1, xp1, dinv16, b1, W2)


def _tc3_body(acc_ref, xp_ref, dinv_ref, b2_ref, batch_ref,
              l1w_ref, l1b_ref, l2w_ref, l2b_ref, out_ref):
    dinv = dinv_ref[:, 0:1]
    agg = acc_ref[0] + acc_ref[1]
    h2 = jnp.maximum(dinv * agg + (dinv * dinv) * xp_ref[...] + b2_ref[...],
                     0.0)
    seg = lax.broadcasted_iota(jnp.int32, (1, G), 1)
    p = (batch_ref[...] == seg).astype(jnp.float32)
    pooled_sum = lax.dot_general(
        p, h2, (((0,), (0,)), ((), ())), preferred_element_type=jnp.float32)
    counts = jnp.sum(p, axis=0)
    pooled = pooled_sum / jnp.maximum(counts, 1.0)[:, None]
    t = jnp.maximum(
        jnp.dot(pooled, l1w_ref[...], preferred_element_type=jnp.float32)
        + l1b_ref[...], 0.0)
    logits = (jnp.dot(t, l2w_ref[...], preferred_element_type=jnp.float32)
              + l2b_ref[...])
    m = jnp.max(logits, axis=1, keepdims=True)
    lse = jnp.log(jnp.sum(jnp.exp(logits - m), axis=1, keepdims=True))
    out_ref[...] = logits - m - lse


def _tc3(acc2, xp2, dinv16, b2, batchp, L1w, L1b, L2w, L2b):
    return pl.pallas_call(
        _tc3_body,
        out_shape=jax.ShapeDtypeStruct((G, C), jnp.float32),
    )(acc2, xp2, dinv16, b2, batchp, L1w, L1b, L2w, L2b)


def kernel(x, edge_index, batch, W1, b1, W2, b2, L1w, L1b, L2w, L2b):
    pad_e = EPAD - E
    srcp = jnp.concatenate(
        [edge_index[0], jnp.full((pad_e,), N, jnp.int32)]
    ).reshape(TOTCH, CH)
    dstp = jnp.concatenate(
        [edge_index[1], jnp.full((pad_e,), N, jnp.int32)]
    ).reshape(TOTCH, CH)
    xpad = jnp.pad(x, ((0, NPAD - N), (0, 0)))
    batchp = jnp.concatenate(
        [batch, jnp.full((NPAD - N,), G, jnp.int32)]).reshape(NPAD, 1)

    degp = _sc_deg(dstp)
    xp1 = _tc0(xpad, W1)
    xs1, dinv16 = _tc1(xp1, degp)
    acc1 = _sc_agg(srcp, dstp, xs1)
    xp2, xs2 = _tc2(acc1, xp1, dinv16, b1.reshape(1, H), W2)
    acc2 = _sc_agg(srcp, dstp, xs2)
    return _tc3(acc2, xp2, dinv16, b2.reshape(1, H), batchp,
                L1w, L1b.reshape(1, 32), L2w, L2b.reshape(1, C))


# speedup vs baseline: 32.9907x; 1.7332x over previous
import functools

import jax
import jax.numpy as jnp
from jax import lax
from jax.experimental import pallas as pl
from jax.experimental.pallas import tpu as pltpu
from jax.experimental.pallas import tpu_sc as plsc

N = 10000
E = 320000
G = 64
D_IN = 128
H = 64
C = 2

NC = 2
NS = 16
NPAD = 10240
EPT = 10240
EPAD = NC * NS * EPT
CH = 128
NCHUNK = EPT // CH
TOTCH = EPAD // CH
N0 = 128
N1 = TOTCH // NS - N0
ZROWS = NPAD // NS

_mesh = plsc.VectorSubcoreMesh(core_axis_name="c", subcore_axis_name="s")
_sc_params = pltpu.CompilerParams(use_tc_tiling_on_sc=False)


def _deg_body(dst_hbm, out_hbm, didx, ones_v, zbuf, acc_sh, sem):
    c = lax.axis_index("c")
    s = lax.axis_index("s")
    wid = c * NS + s
    one16 = jnp.full((16,), 1.0, jnp.float32)
    zero16 = jnp.zeros((16,), jnp.float32)

    def fill(i, _):
        ones_v[i, :] = one16
        return 0
    lax.fori_loop(0, CH, fill, 0)

    def zfill(i, _):
        zbuf[i, :] = zero16
        return 0
    lax.fori_loop(0, ZROWS, zfill, 0)

    pltpu.sync_copy(zbuf, acc_sh.at[pl.ds(s * ZROWS, ZROWS)])
    pltpu.sync_copy(dst_hbm.at[pl.ds(wid * NCHUNK, NCHUNK)], didx)
    plsc.subcore_barrier()

    def step(k, _):
        pltpu.sync_copy(ones_v, acc_sh.at[didx.at[k]], add=True)
        return 0
    lax.fori_loop(0, NCHUNK, step, 0)

    plsc.subcore_barrier()
    rows = pl.ds(s * ZROWS, ZROWS)
    pltpu.sync_copy(acc_sh.at[rows], zbuf)
    pltpu.sync_copy(zbuf, out_hbm.at[c, rows])


@functools.partial(jax.jit, static_argnums=())
def _sc_deg(dstp):
    k = pl.kernel(
        _deg_body,
        out_type=jax.ShapeDtypeStruct((NC, NPAD, 16), jnp.float32),
        mesh=_mesh,
        compiler_params=_sc_params,
        scratch_types=[
            pltpu.VMEM((NCHUNK, CH), jnp.int32),
            pltpu.VMEM((CH, 16), jnp.float32),
            pltpu.VMEM((ZROWS, 16), jnp.float32),
            pltpu.VMEM_SHARED((NPAD, 16), jnp.float32),
            pltpu.SemaphoreType.DMA,
        ],
    )
    return k(dstp)


def _agg_body(src_hbm, dst_hbm, xs_hbm, out_hbm,
              sidx, didx, gb0, gb1, zbuf, acc_sh, xs_sh, sg0, sg1):
    gbs = (gb0, gb1)
    sgs = (sg0, sg1)
    c = lax.axis_index("c")
    s = lax.axis_index("s")
    wid = c * NS + s
    zero16 = jnp.zeros((16,), jnp.float32)

    def zfill(i, _):
        for j in range(4):
            zbuf[i, pl.ds(j * 16, 16)] = zero16
        return 0
    lax.fori_loop(0, CH, zfill, 0)

    def zcopy(i, _):
        rows = pl.ds(s * ZROWS + i * CH, CH)
        pltpu.sync_copy(zbuf, acc_sh.at[rows])
        pltpu.sync_copy(xs_hbm.at[rows], gb0)
        pltpu.sync_copy(gb0, xs_sh.at[rows])
        return 0
    lax.fori_loop(0, ZROWS // CH, zcopy, 0)

    cb = wid * NCHUNK
    pltpu.sync_copy(src_hbm.at[pl.ds(cb, NCHUNK)], sidx)
    pltpu.sync_copy(dst_hbm.at[pl.ds(cb, NCHUNK)], didx)
    plsc.subcore_barrier()

    pltpu.async_copy(xs_sh.at[sidx.at[0]], gb0, sg0)

    def body(g, _):
        k0 = 2 * g
        for b in range(2):
            k = k0 + b
            pltpu.make_async_copy(
                xs_sh.at[sidx.at[k]], gbs[b], sgs[b]).wait()

            @pl.when(k + 1 < NCHUNK)
            def _():
                pltpu.async_copy(xs_sh.at[sidx.at[k + 1]],
                                 gbs[1 - b], sgs[1 - b])

            pltpu.sync_copy(gbs[b], acc_sh.at[didx.at[k]], add=True)
        return 0
    lax.fori_loop(0, NCHUNK // 2, body, 0)

    plsc.subcore_barrier()

    def wb(i, _):
        rows = pl.ds(s * ZROWS + i * CH, CH)
        pltpu.sync_copy(acc_sh.at[rows], zbuf)
        pltpu.sync_copy(zbuf, out_hbm.at[c, rows])
        return 0
    lax.fori_loop(0, ZROWS // CH, wb, 0)


def _sc_agg(srcp, dstp, xs):
    k = pl.kernel(
        _agg_body,
        out_type=jax.ShapeDtypeStruct((NC, NPAD, H), jnp.float32),
        mesh=_mesh,
        compiler_params=_sc_params,
        scratch_types=[
            pltpu.VMEM((NCHUNK, CH), jnp.int32),
            pltpu.VMEM((NCHUNK, CH), jnp.int32),
            pltpu.VMEM((CH, H), jnp.float32),
            pltpu.VMEM((CH, H), jnp.float32),
            pltpu.VMEM((CH, H), jnp.float32),
            pltpu.VMEM_SHARED((NPAD, H), jnp.float32),
            pltpu.VMEM_SHARED((NPAD, H), jnp.float32),
            pltpu.SemaphoreType.DMA,
            pltpu.SemaphoreType.DMA,
        ],
    )
    return k(srcp, dstp, xs)


_BLK = 512
_NBLK = NPAD // _BLK


def _tc0_body(x_ref, w1_ref, xp_ref):
    xp_ref[...] = jnp.dot(x_ref[...], w1_ref[...],
                          preferred_element_type=jnp.float32)


def _tc0(xpad, W1):
    return pl.pallas_call(
        _tc0_body,
        grid=(_NBLK,),
        in_specs=[
            pl.BlockSpec((_BLK, D_IN), lambda i: (i, 0)),
            pl.BlockSpec((D_IN, H), lambda i: (0, 0)),
        ],
        out_specs=pl.BlockSpec((_BLK, H), lambda i: (i, 0)),
        out_shape=jax.ShapeDtypeStruct((NPAD, H), jnp.float32),
    )(xpad, W1)


def _tc1_body(xp_ref, degp_ref, xs_ref, dinv_ref):
    deg = degp_ref[0] + degp_ref[1] + 1.0
    dinv = lax.rsqrt(jnp.maximum(deg, 1.0))
    xs_ref[...] = dinv[:, 0:1] * xp_ref[...]
    dinv_ref[...] = dinv


def _tc1(xp1, degp):
    return pl.pallas_call(
        _tc1_body,
        grid=(_NBLK,),
        in_specs=[
            pl.BlockSpec((_BLK, H), lambda i: (i, 0)),
            pl.BlockSpec((NC, _BLK, 16), lambda i: (0, i, 0)),
        ],
        out_specs=[
            pl.BlockSpec((_BLK, H), lambda i: (i, 0)),
            pl.BlockSpec((_BLK, 16), lambda i: (i, 0)),
        ],
        out_shape=[
            jax.ShapeDtypeStruct((NPAD, H), jnp.float32),
            jax.ShapeDtypeStruct((NPAD, 16), jnp.float32),
        ],
    )(xp1, degp)


def _tc2_body(acc_ref, xp_ref, dinv_ref, b1_ref, w2_ref, xp2_ref, xs2_ref):
    dinv = dinv_ref[:, 0:1]
    agg = acc_ref[0] + acc_ref[1]
    h1 = jnp.maximum(dinv * agg + (dinv * dinv) * xp_ref[...] + b1_ref[...],
                     0.0)
    xp2 = jnp.dot(h1, w2_ref[...], preferred_element_type=jnp.float32)
    xp2_ref[...] = xp2
    xs2_ref[...] = dinv * xp2


def _tc2(acc1, xp1, dinv16, b1, W2):
    return pl.pallas_call(
        _tc2_body,
        grid=(_NBLK,),
        in_specs=[
            pl.BlockSpec((NC, _BLK, H), lambda i: (0, i, 0)),
            pl.BlockSpec((_BLK, H), lambda i: (i, 0)),
            pl.BlockSpec((_BLK, 16), lambda i: (i, 0)),
            pl.BlockSpec((1, H), lambda i: (0, 0)),
            pl.BlockSpec((H, H), lambda i: (0, 0)),
        ],
        out_specs=[
            pl.BlockSpec((_BLK, H), lambda i: (i, 0)),
            pl.BlockSpec((_BLK, H), lambda i: (i, 0)),
        ],
        out_shape=[
            jax.ShapeDtypeStruct((NPAD, H), jnp.float32),
            jax.ShapeDtypeStruct((NPAD, H), jnp.float32),
        ],
    )(acc1, xp1, dinv16, b1, W2)


def _tc3_body(acc_ref, xp_ref, dinv_ref, b2_ref, batch_ref,
              l1w_ref, l1b_ref, l2w_ref, l2b_ref, out_ref):
    dinv = dinv_ref[:, 0:1]
    agg = acc_ref[0] + acc_ref[1]
    h2 = jnp.maximum(dinv * agg + (dinv * dinv) * xp_ref[...] + b2_ref[...],
                     0.0)
    seg = lax.broadcasted_iota(jnp.int32, (1, G), 1)
    p = (batch_ref[...] == seg).astype(jnp.float32)
    pooled_sum = lax.dot_general(
        p, h2, (((0,), (0,)), ((), ())), preferred_element_type=jnp.float32)
    counts = jnp.sum(p, axis=0)
    pooled = pooled_sum / jnp.maximum(counts, 1.0)[:, None]
    t = jnp.maximum(
        jnp.dot(pooled, l1w_ref[...], preferred_element_type=jnp.float32)
        + l1b_ref[...], 0.0)
    logits = (jnp.dot(t, l2w_ref[...], preferred_element_type=jnp.float32)
              + l2b_ref[...])
    m = jnp.max(logits, axis=1, keepdims=True)
    lse = jnp.log(jnp.sum(jnp.exp(logits - m), axis=1, keepdims=True))
    out_ref[...] = logits - m - lse


def _tc3(acc2, xp2, dinv16, b2, batchp, L1w, L1b, L2w, L2b):
    return pl.pallas_call(
        _tc3_body,
        out_shape=jax.ShapeDtypeStruct((G, C), jnp.float32),
    )(acc2, xp2, dinv16, b2, batchp, L1w, L1b, L2w, L2b)


def kernel(x, edge_index, batch, W1, b1, W2, b2, L1w, L1b, L2w, L2b):
    pad_e = EPAD - E
    srcp = jnp.concatenate(
        [edge_index[0], jnp.full((pad_e,), N, jnp.int32)]
    ).reshape(TOTCH, CH)
    dstp = jnp.concatenate(
        [edge_index[1], jnp.full((pad_e,), N, jnp.int32)]
    ).reshape(TOTCH, CH)
    xpad = jnp.pad(x, ((0, NPAD - N), (0, 0)))
    batchp = jnp.concatenate(
        [batch, jnp.full((NPAD - N,), G, jnp.int32)]).reshape(NPAD, 1)

    degp = _sc_deg(dstp)
    xp1 = _tc0(xpad, W1)
    xs1, dinv16 = _tc1(xp1, degp)
    acc1 = _sc_agg(srcp, dstp, xs1)
    xp2, xs2 = _tc2(acc1, xp1, dinv16, b1.reshape(1, H), W2)
    acc2 = _sc_agg(srcp, dstp, xs2)
    return _tc3(acc2, xp2, dinv16, b2.reshape(1, H), batchp,
                L1w, L1b.reshape(1, 32), L2w, L2b.reshape(1, C))
